# baseline (speedup 1.0000x reference)
import jax
from jax.experimental import pallas as pl
from jax.experimental.pallas import tpu as pltpu

ALPHA = 8
NCHUNK = 1
CHUNK = 64


def _pack_body(x_hbm, slow_hbm, fast_hbm, *rest):
    bufs = rest[:NCHUNK]
    in_sems = rest[NCHUNK:2 * NCHUNK]
    out_sem = rest[2 * NCHUNK]
    slow_sem = rest[2 * NCHUNK + 1]

    in_cps = []
    for k in range(NCHUNK):
        cp = pltpu.make_async_copy(
            x_hbm.at[:, pl.ds(CHUNK * k, CHUNK)], bufs[k], in_sems[k])
        cp.start()
        in_cps.append(cp)
    out_cps = []
    for k in range(NCHUNK):
        in_cps[k].wait()
        fast_cp = pltpu.make_async_copy(
            bufs[k], fast_hbm.at[:, pl.ds(CHUNK * k, CHUNK)], out_sem)
        fast_cp.start()
        out_cps.append(fast_cp)
        for j in range(8):
            s = j
            slow_cp = pltpu.make_async_copy(
                bufs[k].at[:, 9 * j], slow_hbm.at[:, s], slow_sem)
            slow_cp.start()
            out_cps.append(slow_cp)
    for cp in out_cps:
        cp.wait()


def kernel(x):
    C, T, H, W = x.shape
    G = T // ALPHA
    slow, fast = pl.pallas_call(
        _pack_body,
        in_specs=[pl.BlockSpec(memory_space=pl.ANY)],
        out_specs=[
            pl.BlockSpec(memory_space=pl.ANY),
            pl.BlockSpec(memory_space=pl.ANY),
        ],
        out_shape=[
            jax.ShapeDtypeStruct((C, G, H, W), x.dtype),
            jax.ShapeDtypeStruct((C, T, H, W), x.dtype),
        ],
        scratch_shapes=(
            [pltpu.VMEM((C, CHUNK, H, W), x.dtype) for _ in range(NCHUNK)]
            + [pltpu.SemaphoreType.DMA for _ in range(NCHUNK)]
            + [pltpu.SemaphoreType.DMA, pltpu.SemaphoreType.DMA]
        ),
    )(x)
    return (slow, fast)


# speedup vs baseline: 1.0593x; 1.0593x over previous
import jax
from jax.experimental import pallas as pl
from jax.experimental.pallas import tpu as pltpu

ALPHA = 8
BOUNDS = (0, 22, 43, 64)
IDX = tuple(9 * s for s in range(8))


def _pack_body(x_hbm, slow_hbm, fast_hbm, *rest):
    nchunk = len(BOUNDS) - 1
    bufs = rest[:nchunk]
    in_sems = rest[nchunk:2 * nchunk]
    out_sem = rest[2 * nchunk]
    slow_sem = rest[2 * nchunk + 1]

    in_cps = []
    for k in range(nchunk):
        lo, hi = BOUNDS[k], BOUNDS[k + 1]
        cp = pltpu.make_async_copy(
            x_hbm.at[:, pl.ds(lo, hi - lo)], bufs[k], in_sems[k])
        cp.start()
        in_cps.append(cp)
    out_cps = []
    for k in range(nchunk):
        lo, hi = BOUNDS[k], BOUNDS[k + 1]
        in_cps[k].wait()
        fast_cp = pltpu.make_async_copy(
            bufs[k], fast_hbm.at[:, pl.ds(lo, hi - lo)], out_sem)
        fast_cp.start()
        out_cps.append(fast_cp)
        for s, t in enumerate(IDX):
            if lo <= t < hi:
                slow_cp = pltpu.make_async_copy(
                    bufs[k].at[:, t - lo], slow_hbm.at[:, s], slow_sem)
                slow_cp.start()
                out_cps.append(slow_cp)
    for cp in out_cps:
        cp.wait()


def kernel(x):
    C, T, H, W = x.shape
    G = T // ALPHA
    slow, fast = pl.pallas_call(
        _pack_body,
        in_specs=[pl.BlockSpec(memory_space=pl.ANY)],
        out_specs=[
            pl.BlockSpec(memory_space=pl.ANY),
            pl.BlockSpec(memory_space=pl.ANY),
        ],
        out_shape=[
            jax.ShapeDtypeStruct((C, G, H, W), x.dtype),
            jax.ShapeDtypeStruct((C, T, H, W), x.dtype),
        ],
        scratch_shapes=(
            [pltpu.VMEM((C, hi - lo, H, W), x.dtype)
             for lo, hi in zip(BOUNDS[:-1], BOUNDS[1:])]
            + [pltpu.SemaphoreType.DMA for _ in range(len(BOUNDS) - 1)]
            + [pltpu.SemaphoreType.DMA, pltpu.SemaphoreType.DMA]
        ),
    )(x)
    return (slow, fast)
